# baseline (speedup 1.0000x reference)
import jax
import jax.numpy as jnp
from jax.experimental import pallas as pl
from jax.experimental.pallas import tpu as pltpu

_TM = 256
_LANE = 128


def _moe_tile(x_ref, w1_ref, b1_ref, w2_ref, b2_ref, ew_ref, eb_ref, o_ref):
    n_e = ew_ref.shape[0]
    x = x_ref[...]
    xb = x.astype(jnp.bfloat16)
    h = jnp.dot(xb, w1_ref[...], preferred_element_type=jnp.float32)
    h = h + b1_ref[...]
    h = h * jax.lax.logistic(h)
    rv = jnp.dot(h, w2_ref[...], preferred_element_type=jnp.float32)
    rv = rv + b2_ref[...]
    col = jax.lax.broadcasted_iota(jnp.int32, rv.shape, 1)
    neg = jnp.float32(-jnp.inf)
    rv = jnp.where(col < n_e, rv, neg)
    m1 = jnp.max(rv, axis=1, keepdims=True)
    i1 = jnp.min(jnp.where(rv == m1, col, _LANE), axis=1, keepdims=True)
    sel1 = col == i1
    rv2 = jnp.where(sel1, neg, rv)
    m2 = jnp.max(rv2, axis=1, keepdims=True)
    i2 = jnp.min(jnp.where(rv2 == m2, col, _LANE), axis=1, keepdims=True)
    sel2 = col == i2
    e2 = jnp.exp(m2 - m1)
    w1v = 1.0 / (1.0 + e2)
    w2v = e2 * w1v
    wd = jnp.where(sel1, w1v, 0.0) + jnp.where(sel2, w2v, 0.0)
    acc = x
    for e in range(n_e):
        eo = jnp.dot(xb, ew_ref[e], preferred_element_type=jnp.float32)
        eo = eo + eb_ref[e][None, :]
        acc = acc + eo * wd[:, e][:, None]
    o_ref[...] = acc


def kernel(x, W1, b1, W2, b2, expert_W, expert_b):
    n_b, n_f, d = x.shape
    n_e = expert_W.shape[0]
    n_tok = n_b * n_f
    x2 = x.reshape(n_tok, d)
    w1b = W1.astype(jnp.bfloat16)
    ewb = expert_W.astype(jnp.bfloat16)
    w2p = jnp.zeros((d, _LANE), W2.dtype).at[:, :n_e].set(W2)
    b2p = jnp.zeros((1, _LANE), b2.dtype).at[:, :n_e].set(b2)
    grid = (n_tok // _TM,)
    out = pl.pallas_call(
        _moe_tile,
        grid=grid,
        in_specs=[
            pl.BlockSpec((_TM, d), lambda t: (t, 0)),
            pl.BlockSpec((d, d), lambda t: (0, 0)),
            pl.BlockSpec((1, d), lambda t: (0, 0)),
            pl.BlockSpec((d, _LANE), lambda t: (0, 0)),
            pl.BlockSpec((1, _LANE), lambda t: (0, 0)),
            pl.BlockSpec((n_e, d, d), lambda t: (0, 0, 0)),
            pl.BlockSpec((n_e, d), lambda t: (0, 0)),
        ],
        out_specs=pl.BlockSpec((_TM, d), lambda t: (t, 0)),
        out_shape=jax.ShapeDtypeStruct((n_tok, d), jnp.float32),
        compiler_params=pltpu.CompilerParams(
            dimension_semantics=("arbitrary",),
        ),
    )(x2, w1b, b1.reshape(1, d), w2p, b2p, ewb, expert_b)
    return out.reshape(n_b, n_f, d)


# speedup vs baseline: 1.0898x; 1.0898x over previous
import jax
import jax.numpy as jnp
from jax.experimental import pallas as pl
from jax.experimental.pallas import tpu as pltpu

_TM = 256
_LANE = 128


def _moe_tile(x_ref, w1_ref, b1_ref, w2_ref, b2_ref, ew_ref, eb_ref, o_ref):
    n_e = ew_ref.shape[0]
    x = x_ref[...]
    xb = x.astype(jnp.bfloat16)
    h = jnp.dot(xb, w1_ref[...].astype(jnp.bfloat16),
                preferred_element_type=jnp.float32)
    h = h + b1_ref[...]
    h = h * jax.lax.logistic(h)
    rv = jnp.dot(h, w2_ref[...], preferred_element_type=jnp.float32)
    rv = rv + b2_ref[...]
    col = jax.lax.broadcasted_iota(jnp.int32, rv.shape, 1)
    neg = jnp.float32(-jnp.inf)
    rv = jnp.where(col < n_e, rv, neg)
    m1 = jnp.max(rv, axis=1, keepdims=True)
    i1 = jnp.min(jnp.where(rv == m1, col, _LANE), axis=1, keepdims=True)
    sel1 = col == i1
    rv2 = jnp.where(sel1, neg, rv)
    m2 = jnp.max(rv2, axis=1, keepdims=True)
    i2 = jnp.min(jnp.where(rv2 == m2, col, _LANE), axis=1, keepdims=True)
    sel2 = col == i2
    e2 = jnp.exp(m2 - m1)
    w1v = 1.0 / (1.0 + e2)
    w2v = e2 * w1v
    wd = jnp.where(sel1, w1v, 0.0) + jnp.where(sel2, w2v, 0.0)
    acc = x
    for e in range(n_e):
        eo = jnp.dot(xb, ew_ref[e].astype(jnp.bfloat16),
                     preferred_element_type=jnp.float32)
        eo = eo + eb_ref[e][None, :]
        acc = acc + eo * wd[:, e][:, None]
    o_ref[...] = acc


def kernel(x, W1, b1, W2, b2, expert_W, expert_b):
    n_b, n_f, d = x.shape
    n_e = expert_W.shape[0]
    n_tok = n_b * n_f
    x2 = x.reshape(n_tok, d)
    w2p = jnp.zeros((d, _LANE), W2.dtype).at[:, :n_e].set(W2)
    b2p = jnp.zeros((1, _LANE), b2.dtype).at[:, :n_e].set(b2)
    grid = (n_tok // _TM,)
    out = pl.pallas_call(
        _moe_tile,
        grid=grid,
        in_specs=[
            pl.BlockSpec((_TM, d), lambda t: (t, 0)),
            pl.BlockSpec((d, d), lambda t: (0, 0)),
            pl.BlockSpec((1, d), lambda t: (0, 0)),
            pl.BlockSpec((d, _LANE), lambda t: (0, 0)),
            pl.BlockSpec((1, _LANE), lambda t: (0, 0)),
            pl.BlockSpec((n_e, d, d), lambda t: (0, 0, 0)),
            pl.BlockSpec((n_e, d), lambda t: (0, 0)),
        ],
        out_specs=pl.BlockSpec((_TM, d), lambda t: (t, 0)),
        out_shape=jax.ShapeDtypeStruct((n_tok, d), jnp.float32),
        compiler_params=pltpu.CompilerParams(
            dimension_semantics=("arbitrary",),
        ),
    )(x2, W1, b1.reshape(1, d), w2p, b2p, expert_W, expert_b)
    return out.reshape(n_b, n_f, d)
